# baseline (speedup 1.0000x reference)
import jax
import jax.numpy as jnp
import numpy as np
from jax import lax
from jax.experimental import pallas as pl
from jax.experimental.pallas import tpu as pltpu
from jax.experimental.pallas import tpu_sc as plsc

N = 10000
E = 160000
D = 256
DW = D // 2
NC, NS = 2, 16
NW = NC * NS
EPW = E // NW
CH = 96
NFULL = EPW // CH
TAIL = EPW - NFULL * CH
CPW = N // NW
CREM = N - CPW * NW

def _avg_rows(dst, srca, srcb, nrows):
  @pl.loop(0, nrows)
  def _row(r):
    for j in range(DW // 16):
      a = plsc.bitcast(srca[r, pl.ds(j * 16, 16)], jnp.bfloat16)
      b = plsc.bitcast(srcb[r, pl.ds(j * 16, 16)], jnp.bfloat16)
      s = (a + b) * 0.5
      lo, hi = plsc.unpack(s, format=plsc.PackFormat.INTERLEAVED,
                           preferred_element_type=jnp.float32)
      dst[r, pl.ds(j * 32, 16)] = lo
      dst[r, pl.ds(j * 32 + 16, 16)] = hi


def _body(table, tbl16, idx0, idx1, out,
          idxv0, idxv1, r0a, r1a, fa, r0b, r1b, fb, gsa, gsb, ssa, ssb, csem):
  wid = lax.axis_index("s") * NC + lax.axis_index("c")
  base = wid * EPW

  cb = wid * CPW
  cpy = pltpu.async_copy(table.at[pl.ds(cb, CPW)], out.at[pl.ds(cb, CPW)], csem)

  pltpu.sync_copy(idx0.at[pl.ds(base, EPW)], idxv0)
  pltpu.sync_copy(idx1.at[pl.ds(base, EPW)], idxv1)

  rows0 = (r0a, r0b)
  rows1 = (r1a, r1b)
  frows = (fa, fb)
  gs = (gsa, gsb)
  ss = (ssa, ssb)

  pltpu.async_copy(tbl16.at[idxv0.at[pl.ds(0, CH)]], r0a, gsa)
  pltpu.async_copy(tbl16.at[idxv1.at[pl.ds(0, CH)]], r1a, gsa)

  @pl.loop(0, NFULL, step=2)
  def _super(i):
    for b in range(2):
      c = i + b

      pltpu.make_async_copy(tbl16.at[pl.ds(0, CH)], rows0[b], gs[b]).wait()
      pltpu.make_async_copy(tbl16.at[pl.ds(0, CH)], rows1[b], gs[b]).wait()

      @pl.when(c >= 1)
      def _wait_store():
        pltpu.make_async_copy(frows[1 - b], out.at[pl.ds(N, CH)],
                              ss[1 - b]).wait()

      @pl.when(c + 1 < NFULL)
      def _next_gather():
        off = (c + 1) * CH
        pltpu.async_copy(tbl16.at[idxv0.at[pl.ds(off, CH)]], rows0[1 - b],
                         gs[1 - b])
        pltpu.async_copy(tbl16.at[idxv1.at[pl.ds(off, CH)]], rows1[1 - b],
                         gs[1 - b])

      _avg_rows(frows[b], rows0[b], rows1[b], CH)
      pltpu.async_copy(frows[b], out.at[pl.ds(N + base + c * CH, CH)], ss[b])

  toff = NFULL * CH
  t0 = pltpu.async_copy(tbl16.at[idxv0.at[pl.ds(toff, TAIL)]],
                        r0a.at[pl.ds(0, TAIL)], gsa)
  t1 = pltpu.async_copy(tbl16.at[idxv1.at[pl.ds(toff, TAIL)]],
                        r1a.at[pl.ds(0, TAIL)], gsa)
  pltpu.make_async_copy(fb, out.at[pl.ds(N, CH)], ssb).wait()
  t0.wait()
  t1.wait()
  _avg_rows(fa, r0a, r1a, TAIL)
  pltpu.sync_copy(fa.at[pl.ds(0, TAIL)], out.at[pl.ds(N + base + toff, TAIL)])

  @pl.when(wid < CREM)
  def _rem():
    pltpu.sync_copy(table.at[pl.ds(CPW * NW + wid, 1)],
                    out.at[pl.ds(CPW * NW + wid, 1)])

  cpy.wait()


_mesh = plsc.VectorSubcoreMesh(core_axis_name="c", subcore_axis_name="s")

_k = pl.kernel(
    _body,
    out_type=jax.ShapeDtypeStruct((N + E, D), jnp.float32),
    mesh=_mesh,
    compiler_params=pltpu.CompilerParams(needs_layout_passes=False),
    scratch_types=[
        pltpu.VMEM((EPW,), jnp.int32),
        pltpu.VMEM((EPW,), jnp.int32),
        pltpu.VMEM((CH, DW), jnp.uint32),
        pltpu.VMEM((CH, DW), jnp.uint32),
        pltpu.VMEM((CH, D), jnp.float32),
        pltpu.VMEM((CH, DW), jnp.uint32),
        pltpu.VMEM((CH, DW), jnp.uint32),
        pltpu.VMEM((CH, D), jnp.float32),
        pltpu.SemaphoreType.DMA,
        pltpu.SemaphoreType.DMA,
        pltpu.SemaphoreType.DMA,
        pltpu.SemaphoreType.DMA,
        pltpu.SemaphoreType.DMA,
    ],
)


@jax.jit
def kernel(inputs, unpool_idx):
  table = inputs[0]
  t16 = table.astype(jnp.bfloat16).reshape(N, D // 32, 2, 16)
  t16 = t16.swapaxes(2, 3).reshape(N, DW, 2)
  t16u = lax.bitcast_convert_type(t16, jnp.uint32)
  idx = unpool_idx.astype(jnp.int32)
  out = _k(table, t16u, idx[:, 0], idx[:, 1])
  return out[None]


# speedup vs baseline: 1.1624x; 1.1624x over previous
import jax
import jax.numpy as jnp
from jax import lax
from jax.experimental import pallas as pl
from jax.experimental.pallas import tpu as pltpu
from jax.experimental.pallas import tpu_sc as plsc

N = 10000
E = 160000
D = 256
NC, NS = 2, 16
NW = NC * NS
EPW = E // NW
CH = 72
NB = 3
NFULL = EPW // CH
TAIL = EPW - NFULL * CH
CPW = N // NW
CREM = N - CPW * NW


def _avg_rows(dst, src, nrows):
  @pl.loop(0, nrows)
  def _row(r):
    for j in range(D // 16):
      sl = pl.ds(j * 16, 16)
      dst[r, sl] = (dst[r, sl] + src[r, sl]) * 0.5


def _body(table, idx0, idx1, out,
          idxv0, idxv1, r0a, r1a, r0b, r1b, r0c, r1c,
          gsa, gsb, gsc, ssa, ssb, ssc, csem):
  wid = lax.axis_index("s") * NC + lax.axis_index("c")
  base = wid * EPW

  cb = wid * CPW
  cpy = pltpu.async_copy(table.at[pl.ds(cb, CPW)], out.at[pl.ds(cb, CPW)], csem)

  pltpu.sync_copy(idx0.at[pl.ds(base, EPW)], idxv0)
  pltpu.sync_copy(idx1.at[pl.ds(base, EPW)], idxv1)

  rows0 = (r0a, r0b, r0c)
  rows1 = (r1a, r1b, r1c)
  gs = (gsa, gsb, gsc)
  ss = (ssa, ssb, ssc)

  for p in range(2):
    pltpu.async_copy(table.at[idxv0.at[pl.ds(p * CH, CH)]], rows0[p], gs[p])
    pltpu.async_copy(table.at[idxv1.at[pl.ds(p * CH, CH)]], rows1[p], gs[p])

  @pl.loop(0, NFULL, step=NB)
  def _super(i):
    for b in range(NB):
      c = i + b

      pltpu.make_async_copy(table.at[pl.ds(0, CH)], rows0[b], gs[b]).wait()
      pltpu.make_async_copy(table.at[pl.ds(0, CH)], rows1[b], gs[b]).wait()

      nxt = (b + 2) % NB

      @pl.when(c >= 1)
      def _wait_store():
        pltpu.make_async_copy(rows0[nxt], out.at[pl.ds(N, CH)],
                              ss[nxt]).wait()

      @pl.when(c + 2 < NFULL)
      def _next_gather():
        off = (c + 2) * CH
        pltpu.async_copy(table.at[idxv0.at[pl.ds(off, CH)]], rows0[nxt],
                         gs[nxt])
        pltpu.async_copy(table.at[idxv1.at[pl.ds(off, CH)]], rows1[nxt],
                         gs[nxt])

      _avg_rows(rows0[b], rows1[b], CH)
      pltpu.async_copy(rows0[b], out.at[pl.ds(N + base + c * CH, CH)], ss[b])

  toff = NFULL * CH
  t0 = pltpu.async_copy(table.at[idxv0.at[pl.ds(toff, TAIL)]],
                        r0a.at[pl.ds(0, TAIL)], gsa)
  t1 = pltpu.async_copy(table.at[idxv1.at[pl.ds(toff, TAIL)]],
                        r1a.at[pl.ds(0, TAIL)], gsa)
  pltpu.make_async_copy(r0c, out.at[pl.ds(N, CH)], ssc).wait()
  t0.wait()
  t1.wait()
  _avg_rows(r0a, r1a, TAIL)
  pltpu.sync_copy(r0a.at[pl.ds(0, TAIL)], out.at[pl.ds(N + base + toff, TAIL)])

  @pl.when(wid < CREM)
  def _rem():
    pltpu.sync_copy(table.at[pl.ds(CPW * NW + wid, 1)],
                    out.at[pl.ds(CPW * NW + wid, 1)])

  cpy.wait()


_mesh = plsc.VectorSubcoreMesh(core_axis_name="c", subcore_axis_name="s")

_k = pl.kernel(
    _body,
    out_type=jax.ShapeDtypeStruct((N + E, D), jnp.float32),
    mesh=_mesh,
    scratch_types=[
        pltpu.VMEM((EPW,), jnp.int32),
        pltpu.VMEM((EPW,), jnp.int32),
        pltpu.VMEM((CH, D), jnp.float32),
        pltpu.VMEM((CH, D), jnp.float32),
        pltpu.VMEM((CH, D), jnp.float32),
        pltpu.VMEM((CH, D), jnp.float32),
        pltpu.VMEM((CH, D), jnp.float32),
        pltpu.VMEM((CH, D), jnp.float32),
        pltpu.SemaphoreType.DMA,
        pltpu.SemaphoreType.DMA,
        pltpu.SemaphoreType.DMA,
        pltpu.SemaphoreType.DMA,
        pltpu.SemaphoreType.DMA,
        pltpu.SemaphoreType.DMA,
        pltpu.SemaphoreType.DMA,
    ],
)


@jax.jit
def kernel(inputs, unpool_idx):
  table = inputs[0]
  idx = unpool_idx.astype(jnp.int32)
  out = _k(table, idx[:, 0], idx[:, 1])
  return out[None]
